# baseline (speedup 1.0000x reference)
import jax
import jax.numpy as jnp
from jax import lax
from jax.experimental import pallas as pl
from jax.experimental.pallas import tpu as pltpu
from jax.experimental.pallas import tpu_sc as plsc

_B = 128
_F0 = 10000
_H = 4
_E0 = 80000
_N1 = 2000
_IN1 = _N1 * _H
_E1 = 32000
_N2 = 512
_C = 128

_NC = 2
_NS = 16
_NW = _NC * _NS
_BPW = _B // _NW
_L = 16

_CHUNK = 800
_CCHUNK = 1000


def _sc_body(data_hbm, src0_hbm, dst0_hbm, src1_hbm, dst1_hbm,
             p0t_hbm, p1t_hbm, min0_hbm, max0_hbm, min1_hbm, max1_hbm,
             out_hbm, cnt0, acc1, cnt1, srcb0, srcb1, dstb0, dstb1,
             sem0, sem1, sem2, sem3, rta, rtb, tsc, shared):
    sbufs = (srcb0, srcb1)
    dbufs = (dstb0, dstb1)
    sems = (sem0, sem1, sem2, sem3)
    cid = lax.axis_index("c")
    sid = lax.axis_index("s")
    wid = sid * _NC + cid

    iota = lax.iota(jnp.int32, _L)
    hconst = lax.shift_right_logical(iota, 2)
    bconst = lax.bitwise_and(iota, 3)
    hconst4 = hconst + 4
    iota17 = iota * 17
    zeros = jnp.zeros((_L,), jnp.float32)
    ones = jnp.ones((_L,), jnp.float32)
    lane0 = iota == 0

    def zero_f32(ref, n):
        def body(i, _):
            ref[pl.ds(i * _L, _L)] = zeros
            return 0
        lax.fori_loop(0, n // _L, body, 0)

    zero_f32(cnt0, 2048)
    zero_f32(acc1, _N2 * _L)
    zero_f32(cnt1, _N2)

    def edge_pass(src_hbm, dst_hbm, n_edges, in_dim, table, params, acc):
        nchunk = n_edges // _CHUNK

        def start(c, k):
            pltpu.async_copy(src_hbm.at[pl.ds(c * _CHUNK, _CHUNK)],
                             sbufs[k], sems[2 * k])
            pltpu.async_copy(dst_hbm.at[pl.ds(c * _CHUNK, _CHUNK)],
                             dbufs[k], sems[2 * k + 1])

        def wait(k):
            pltpu.make_async_copy(src_hbm.at[pl.ds(0, _CHUNK)], sbufs[k],
                                  sems[2 * k]).wait()
            pltpu.make_async_copy(dst_hbm.at[pl.ds(0, _CHUNK)], dbufs[k],
                                  sems[2 * k + 1]).wait()

        start(0, 0)

        def chunk_body(c, _):
            k = lax.rem(c, 2)

            @pl.when(jnp.logical_and(c + 1 < nchunk, k == 0))
            def _():
                start(c + 1, 1)

            @pl.when(jnp.logical_and(c + 1 < nchunk, k == 1))
            def _():
                start(c + 1, 0)

            def work(sb, db):
                def one_group(g, toff):
                    sv = sb[pl.ds(g * _L, _L)]
                    dv16 = db[pl.ds(g * _L, _L)] * _L
                    avs = [plsc.load_gather(table, [sv + b * in_dim])
                           for b in range(4)]
                    pvs = [plsc.load_gather(params, [sv + h * in_dim])
                           for h in range(4)]
                    for h in range(4):
                        for b in range(4):
                            plsc.store_scatter(
                                tsc, [iota17 + (toff + h * 4 + b)],
                                avs[b] * pvs[h])
                    for half in range(2):
                        rows = [plsc.load_gather(
                                    tsc, [iota + 17 * (half * 8 + j) + toff])
                                for j in range(8)]
                        for j in range(8):
                            l = half * 8 + j
                            addr = jnp.full((_L,), dv16[l], jnp.int32) + iota
                            plsc.addupdate_scatter(acc, [addr], rows[j])

                def group_body(g2, _):
                    one_group(g2 * 2, 0)
                    one_group(g2 * 2 + 1, 272)
                    return 0
                lax.fori_loop(0, _CHUNK // _L // 2, group_body, 0)

            @pl.when(k == 0)
            def _():
                wait(0)
                work(sbufs[0], dbufs[0])

            @pl.when(k == 1)
            def _():
                wait(1)
                work(sbufs[1], dbufs[1])
            return 0

        lax.fori_loop(0, nchunk, chunk_body, 0)

    def count_pass(dst_hbm, n_edges, cnt, n_pad):
        per = n_edges // _NS
        blk = n_pad // _NS

        def chunk_body(c, _):
            pltpu.sync_copy(
                dst_hbm.at[pl.ds(sid * per + c * _CCHUNK, _CCHUNK)],
                dstb0.at[pl.ds(0, _CCHUNK)])

            def group_body(g, _):
                dv = dstb0[pl.ds(g * _L, _L)]
                plsc.addupdate_scatter(cnt, [dv], ones)
                return 0

            lax.fori_loop(0, _CCHUNK // _L, group_body, 0)
            return 0

        lax.fori_loop(0, per // _CCHUNK, chunk_body, 0)

        pltpu.sync_copy(cnt, shared.at[pl.ds(sid * 2048, n_pad)])
        plsc.subcore_barrier()
        base = pl.multiple_of(sid * blk, 8)
        zero_f32(rta, _L * (blk // _L))
        for t in range(_NS):
            pltpu.sync_copy(shared.at[pl.ds(t * 2048 + base, blk)],
                            rtb.at[pl.ds(0, blk)])
            def add_body(j, _):
                rta[pl.ds(j * _L, _L)] = (rta[pl.ds(j * _L, _L)]
                                          + rtb[pl.ds(j * _L, _L)])
                return 0
            lax.fori_loop(0, blk // _L, add_body, 0)
        pltpu.sync_copy(rta.at[pl.ds(0, blk)],
                        shared.at[pl.ds(_NS * 2048 + base, blk)])
        plsc.subcore_barrier()
        pltpu.sync_copy(shared.at[pl.ds(_NS * 2048, n_pad)],
                        cnt.at[pl.ds(0, n_pad)])
        plsc.subcore_barrier()

    def postprocess(acc, cnt, n_out, mnv, mxv):
        def rc_body(j, _):
            c = cnt[pl.ds(j * _L, _L)]
            cnt[pl.ds(j * _L, _L)] = 1.0 / jnp.maximum(c, 1.0)
            return 0
        lax.fori_loop(0, n_out // _L, rc_body, 0)

        def sc_body(i, _):
            fb = jnp.full((_L,), i * _L, jnp.int32) + iota
            rc = plsc.load_gather(cnt, [lax.shift_right_logical(fb, 2)])
            mn = mnv[pl.ds(i * _L, _L)]
            mx = mxv[pl.ds(i * _L, _L)]
            inv = 1.0 / (mx - mn + 1e-8)
            mxv[pl.ds(i * _L, _L)] = rc * inv
            mnv[pl.ds(i * _L, _L)] = mn * inv
            return 0
        lax.fori_loop(0, n_out * _H // _L, sc_body, 0)

        def row_body(n2, _):
            row0 = acc[pl.ds(n2 * 32, _L)]
            row1 = acc[pl.ds(n2 * 32 + _L, _L)]
            slx = mxv.at[pl.ds(n2 * 8, _L)]
            sln = mnv.at[pl.ds(n2 * 8, _L)]
            s0 = plsc.load_gather(slx, [hconst])
            s1 = plsc.load_gather(slx, [hconst4])
            o0 = plsc.load_gather(sln, [hconst])
            o1 = plsc.load_gather(sln, [hconst4])
            acc[pl.ds(n2 * 32, _L)] = jnp.maximum(row0 * s0 - o0, 0.0)
            acc[pl.ds(n2 * 32 + _L, _L)] = jnp.maximum(row1 * s1 - o1, 0.0)
            return 0
        lax.fori_loop(0, n_out // 2, row_body, 0)

    def main(acc0):
        zero_f32(acc0, _N1 * _L)

        def layer0(dataT, p0v):
            cpa = pltpu.async_copy(
                data_hbm.at[pl.ds(wid * (_BPW * _F0), _BPW * _F0)], dataT,
                sem0)
            cpb = pltpu.async_copy(p0t_hbm, p0v, sem1)
            count_pass(dst0_hbm, _E0, cnt0, 2048)
            cpa.wait()
            cpb.wait()
            edge_pass(src0_hbm, dst0_hbm, _E0, _F0, dataT, p0v, acc0)

        pl.run_scoped(layer0,
                      pltpu.VMEM((_BPW * _F0,), jnp.float32),
                      pltpu.VMEM((_H * _F0,), jnp.float32))

        def post0(min0v, max0v):
            pltpu.sync_copy(min0_hbm, min0v.at[pl.ds(0, _IN1)])
            pltpu.sync_copy(max0_hbm, max0v.at[pl.ds(0, _IN1)])
            postprocess(acc0, cnt0, _N1, min0v, max0v)

        pl.run_scoped(post0,
                      pltpu.VMEM((_IN1 + _L,), jnp.float32),
                      pltpu.VMEM((_IN1 + _L,), jnp.float32))

        def layer1(t1b, p1v):
            cpa = pltpu.async_copy(p1t_hbm, p1v, sem0)

            def t1_build(j, _):
                w = jnp.full((_L,), j * _L, jnp.int32) + iota
                v = acc0[pl.ds(j * _L, _L)]
                aw = lax.bitwise_and(w, 3) * _IN1 + lax.shift_right_logical(w, 2)
                plsc.store_scatter(t1b, [aw], v)
                return 0
            lax.fori_loop(0, _IN1 * 4 // _L, t1_build, 0)

            count_pass(dst1_hbm, _E1, cnt1, _N2)
            cpa.wait()
            edge_pass(src1_hbm, dst1_hbm, _E1, _IN1, t1b, p1v, acc1)

        pl.run_scoped(layer1,
                      pltpu.VMEM((_BPW * _IN1,), jnp.float32),
                      pltpu.VMEM((_H * _IN1,), jnp.float32))

    pl.run_scoped(main, pltpu.VMEM((_N1 * _L,), jnp.float32))

    def post1(min1v, max1v):
        pltpu.sync_copy(min1_hbm, min1v.at[pl.ds(0, _N2 * _H)])
        pltpu.sync_copy(max1_hbm, max1v.at[pl.ds(0, _N2 * _H)])
        postprocess(acc1, cnt1, _N2, min1v, max1v)

    pl.run_scoped(post1,
                  pltpu.VMEM((_N2 * _H + _L,), jnp.float32),
                  pltpu.VMEM((_N2 * _H + _L,), jnp.float32))

    def emit(outbuf):
        def body(j, _):
            b = lax.shift_right_logical(j, 7)
            fb = lax.bitwise_and(j, 127) * _L
            idx = (jnp.full((_L,), fb, jnp.int32) + iota) * 4 + b
            outbuf[pl.ds(j * _L, _L)] = plsc.load_gather(acc1, [idx])
            return 0
        lax.fori_loop(0, _BPW * (_N2 * _H // _L), body, 0)
        pltpu.sync_copy(
            outbuf, out_hbm.at[pl.ds(wid * (_BPW * _N2 * _H),
                                     _BPW * _N2 * _H)])

    pl.run_scoped(emit, pltpu.VMEM((_BPW * _N2 * _H,), jnp.float32))


@jax.jit
def _gnn_sc(data_f, src0, dst0, src1, dst1, p0t, p1t, min0, max0, min1,
            max1):
    mesh = plsc.VectorSubcoreMesh(core_axis_name="c", subcore_axis_name="s",
                                  num_cores=_NC, num_subcores=_NS)
    f = pl.kernel(
        _sc_body,
        out_type=jax.ShapeDtypeStruct((_B * _N2 * _H,), jnp.float32),
        mesh=mesh,
        compiler_params=pltpu.CompilerParams(needs_layout_passes=False),
        scratch_types=[
            pltpu.VMEM((2048,), jnp.float32),
            pltpu.VMEM((_N2 * _L,), jnp.float32),
            pltpu.VMEM((_N2,), jnp.float32),
            pltpu.VMEM((_CHUNK,), jnp.int32),
            pltpu.VMEM((_CHUNK,), jnp.int32),
            pltpu.VMEM((_CHUNK,), jnp.int32),
            pltpu.VMEM((_CHUNK,), jnp.int32),
            pltpu.SemaphoreType.DMA,
            pltpu.SemaphoreType.DMA,
            pltpu.SemaphoreType.DMA,
            pltpu.SemaphoreType.DMA,
            pltpu.VMEM((128,), jnp.float32),
            pltpu.VMEM((128,), jnp.float32),
            pltpu.VMEM((_L * 17 * 2,), jnp.float32),
            pltpu.VMEM_SHARED(((_NS + 1) * 2048,), jnp.float32),
        ],
    )
    return f(data_f, src0, dst0, src1, dst1, p0t, p1t, min0, max0, min1,
             max1)


def _mm_body(h_ref, w_ref, b_ref, o_ref):
    o_ref[...] = lax.dot_general(
        h_ref[...], w_ref[...], (((1,), (1,)), ((), ())),
        preferred_element_type=jnp.float32) + b_ref[...]


@jax.jit
def _out_proj(h1, w_out, b_out2d):
    return pl.pallas_call(
        _mm_body,
        out_shape=jax.ShapeDtypeStruct((_B, _C), jnp.float32),
    )(h1, w_out, b_out2d)


def kernel(data, edge_index0, edge_index1, params0, params1,
           min0, max0, min1, max1, W_out, b_out):
    src0 = edge_index0[0].astype(jnp.int32)
    dst0 = edge_index0[1].astype(jnp.int32)
    src1 = edge_index1[0].astype(jnp.int32)
    dst1 = edge_index1[1].astype(jnp.int32)
    h1_flat = _gnn_sc(data.reshape(-1), src0, dst0, src1, dst1,
                      params0.T.reshape(-1), params1.T.reshape(-1),
                      min0, max0, min1, max1)
    h1 = h1_flat.reshape(_B, _N2 * _H)
    return _out_proj(h1, W_out, b_out.reshape(1, _C))


# speedup vs baseline: 1.0549x; 1.0549x over previous
import jax
import jax.numpy as jnp
from jax import lax
from jax.experimental import pallas as pl
from jax.experimental.pallas import tpu as pltpu
from jax.experimental.pallas import tpu_sc as plsc

_B = 128
_F0 = 10000
_H = 4
_E0 = 80000
_N1 = 2000
_IN1 = _N1 * _H
_E1 = 32000
_N2 = 512
_C = 128

_NC = 2
_NS = 16
_NW = _NC * _NS
_BPW = _B // _NW
_L = 16

_CHUNK = 800
_CCHUNK = 1000


def _sc_body(data_hbm, src0_hbm, dst0_hbm, src1_hbm, dst1_hbm,
             p0t_hbm, p1t_hbm, min0_hbm, max0_hbm, min1_hbm, max1_hbm,
             out_hbm, cnt0, acc1, cnt1, srcb0, srcb1, dstb0, dstb1,
             sem0, sem1, sem2, sem3, rta, rtb, tsc, shared):
    sbufs = (srcb0, srcb1)
    dbufs = (dstb0, dstb1)
    sems = (sem0, sem1, sem2, sem3)
    cid = lax.axis_index("c")
    sid = lax.axis_index("s")
    wid = sid * _NC + cid

    iota = lax.iota(jnp.int32, _L)
    hconst = lax.shift_right_logical(iota, 2)
    bconst = lax.bitwise_and(iota, 3)
    hconst4 = hconst + 4
    iota17 = iota * 17
    zeros = jnp.zeros((_L,), jnp.float32)
    ones = jnp.ones((_L,), jnp.float32)
    lane0 = iota == 0

    def zero_f32(ref, n):
        def body(i, _):
            ref[pl.ds(i * _L, _L)] = zeros
            return 0
        lax.fori_loop(0, n // _L, body, 0)

    zero_f32(cnt0, 2048)
    zero_f32(acc1, _N2 * _L)
    zero_f32(cnt1, _N2)

    def edge_pass(src_hbm, dst_hbm, n_edges, in_dim, table, params, acc):
        nchunk = n_edges // _CHUNK

        def start(c, k):
            pltpu.async_copy(src_hbm.at[pl.ds(c * _CHUNK, _CHUNK)],
                             sbufs[k], sems[2 * k])
            pltpu.async_copy(dst_hbm.at[pl.ds(c * _CHUNK, _CHUNK)],
                             dbufs[k], sems[2 * k + 1])

        def wait(k):
            pltpu.make_async_copy(src_hbm.at[pl.ds(0, _CHUNK)], sbufs[k],
                                  sems[2 * k]).wait()
            pltpu.make_async_copy(dst_hbm.at[pl.ds(0, _CHUNK)], dbufs[k],
                                  sems[2 * k + 1]).wait()

        start(0, 0)

        def chunk_body(c, _):
            k = lax.rem(c, 2)

            @pl.when(jnp.logical_and(c + 1 < nchunk, k == 0))
            def _():
                start(c + 1, 1)

            @pl.when(jnp.logical_and(c + 1 < nchunk, k == 1))
            def _():
                start(c + 1, 0)

            def work(sb, db):
                def group_body(g, _):
                    sv = sb[pl.ds(g * _L, _L)]
                    dv16 = db[pl.ds(g * _L, _L)] * _L
                    avs = [plsc.load_gather(table, [sv + b * in_dim])
                           for b in range(4)]
                    pvs = [plsc.load_gather(params, [sv + h * in_dim])
                           for h in range(4)]
                    for h in range(4):
                        for b in range(4):
                            plsc.store_scatter(
                                tsc, [iota17 + (h * 4 + b)],
                                avs[b] * pvs[h])
                    for half in range(2):
                        rows = [plsc.load_gather(tsc,
                                                 [iota + 17 * (half * 8 + j)])
                                for j in range(8)]
                        for j in range(8):
                            l = half * 8 + j
                            addr = jnp.full((_L,), dv16[l], jnp.int32) + iota
                            plsc.addupdate_scatter(acc, [addr], rows[j])
                    return 0
                lax.fori_loop(0, _CHUNK // _L, group_body, 0)

            @pl.when(k == 0)
            def _():
                wait(0)
                work(sbufs[0], dbufs[0])

            @pl.when(k == 1)
            def _():
                wait(1)
                work(sbufs[1], dbufs[1])
            return 0

        lax.fori_loop(0, nchunk, chunk_body, 0)

    def count_pass(dst_hbm, n_edges, cnt, n_pad):
        per = n_edges // _NS
        blk = n_pad // _NS

        def chunk_body(c, _):
            pltpu.sync_copy(
                dst_hbm.at[pl.ds(sid * per + c * _CCHUNK, _CCHUNK)],
                dstb0.at[pl.ds(0, _CCHUNK)])

            def group_body(g, _):
                dv = dstb0[pl.ds(g * _L, _L)]
                plsc.addupdate_scatter(cnt, [dv], ones)
                return 0

            lax.fori_loop(0, _CCHUNK // _L, group_body, 0)
            return 0

        lax.fori_loop(0, per // _CCHUNK, chunk_body, 0)

        pltpu.sync_copy(cnt, shared.at[pl.ds(sid * 2048, n_pad)])
        plsc.subcore_barrier()
        base = pl.multiple_of(sid * blk, 8)
        zero_f32(rta, _L * (blk // _L))
        for t in range(_NS):
            pltpu.sync_copy(shared.at[pl.ds(t * 2048 + base, blk)],
                            rtb.at[pl.ds(0, blk)])
            def add_body(j, _):
                rta[pl.ds(j * _L, _L)] = (rta[pl.ds(j * _L, _L)]
                                          + rtb[pl.ds(j * _L, _L)])
                return 0
            lax.fori_loop(0, blk // _L, add_body, 0)
        pltpu.sync_copy(rta.at[pl.ds(0, blk)],
                        shared.at[pl.ds(_NS * 2048 + base, blk)])
        plsc.subcore_barrier()
        pltpu.sync_copy(shared.at[pl.ds(_NS * 2048, n_pad)],
                        cnt.at[pl.ds(0, n_pad)])
        plsc.subcore_barrier()

    def postprocess(acc, cnt, n_out, mnv, mxv):
        def rc_body(j, _):
            c = cnt[pl.ds(j * _L, _L)]
            cnt[pl.ds(j * _L, _L)] = 1.0 / jnp.maximum(c, 1.0)
            return 0
        lax.fori_loop(0, n_out // _L, rc_body, 0)

        def sc_body(i, _):
            fb = jnp.full((_L,), i * _L, jnp.int32) + iota
            rc = plsc.load_gather(cnt, [lax.shift_right_logical(fb, 2)])
            mn = mnv[pl.ds(i * _L, _L)]
            mx = mxv[pl.ds(i * _L, _L)]
            inv = 1.0 / (mx - mn + 1e-8)
            mxv[pl.ds(i * _L, _L)] = rc * inv
            mnv[pl.ds(i * _L, _L)] = mn * inv
            return 0
        lax.fori_loop(0, n_out * _H // _L, sc_body, 0)

        def row_body(n2, _):
            row0 = acc[pl.ds(n2 * 32, _L)]
            row1 = acc[pl.ds(n2 * 32 + _L, _L)]
            slx = mxv.at[pl.ds(n2 * 8, _L)]
            sln = mnv.at[pl.ds(n2 * 8, _L)]
            s0 = plsc.load_gather(slx, [hconst])
            s1 = plsc.load_gather(slx, [hconst4])
            o0 = plsc.load_gather(sln, [hconst])
            o1 = plsc.load_gather(sln, [hconst4])
            acc[pl.ds(n2 * 32, _L)] = jnp.maximum(row0 * s0 - o0, 0.0)
            acc[pl.ds(n2 * 32 + _L, _L)] = jnp.maximum(row1 * s1 - o1, 0.0)
            return 0
        lax.fori_loop(0, n_out // 2, row_body, 0)

    def main(acc0):
        zero_f32(acc0, _N1 * _L)

        def layer0(dataT, p0v):
            cpa = pltpu.async_copy(
                data_hbm.at[pl.ds(wid * (_BPW * _F0), _BPW * _F0)], dataT,
                sem0)
            cpb = pltpu.async_copy(p0t_hbm, p0v, sem1)
            count_pass(dst0_hbm, _E0, cnt0, 2048)
            cpa.wait()
            cpb.wait()
            edge_pass(src0_hbm, dst0_hbm, _E0, _F0, dataT, p0v, acc0)

        pl.run_scoped(layer0,
                      pltpu.VMEM((_BPW * _F0,), jnp.float32),
                      pltpu.VMEM((_H * _F0,), jnp.float32))

        def post0(min0v, max0v):
            pltpu.sync_copy(min0_hbm, min0v.at[pl.ds(0, _IN1)])
            pltpu.sync_copy(max0_hbm, max0v.at[pl.ds(0, _IN1)])
            postprocess(acc0, cnt0, _N1, min0v, max0v)

        pl.run_scoped(post0,
                      pltpu.VMEM((_IN1 + _L,), jnp.float32),
                      pltpu.VMEM((_IN1 + _L,), jnp.float32))

        def layer1(t1b, p1v):
            cpa = pltpu.async_copy(p1t_hbm, p1v, sem0)

            def t1_build(j, _):
                w = jnp.full((_L,), j * _L, jnp.int32) + iota
                v = acc0[pl.ds(j * _L, _L)]
                aw = lax.bitwise_and(w, 3) * _IN1 + lax.shift_right_logical(w, 2)
                plsc.store_scatter(t1b, [aw], v)
                return 0
            lax.fori_loop(0, _IN1 * 4 // _L, t1_build, 0)

            count_pass(dst1_hbm, _E1, cnt1, _N2)
            cpa.wait()
            edge_pass(src1_hbm, dst1_hbm, _E1, _IN1, t1b, p1v, acc1)

        pl.run_scoped(layer1,
                      pltpu.VMEM((_BPW * _IN1,), jnp.float32),
                      pltpu.VMEM((_H * _IN1,), jnp.float32))

    pl.run_scoped(main, pltpu.VMEM((_N1 * _L,), jnp.float32))

    def post1(min1v, max1v):
        pltpu.sync_copy(min1_hbm, min1v.at[pl.ds(0, _N2 * _H)])
        pltpu.sync_copy(max1_hbm, max1v.at[pl.ds(0, _N2 * _H)])
        postprocess(acc1, cnt1, _N2, min1v, max1v)

    pl.run_scoped(post1,
                  pltpu.VMEM((_N2 * _H + _L,), jnp.float32),
                  pltpu.VMEM((_N2 * _H + _L,), jnp.float32))

    def emit(outbuf):
        def body(j, _):
            b = lax.shift_right_logical(j, 7)
            fb = lax.bitwise_and(j, 127) * _L
            idx = (jnp.full((_L,), fb, jnp.int32) + iota) * 4 + b
            outbuf[pl.ds(j * _L, _L)] = plsc.load_gather(acc1, [idx])
            return 0
        lax.fori_loop(0, _BPW * (_N2 * _H // _L), body, 0)
        pltpu.sync_copy(
            outbuf, out_hbm.at[pl.ds(wid * (_BPW * _N2 * _H),
                                     _BPW * _N2 * _H)])

    pl.run_scoped(emit, pltpu.VMEM((_BPW * _N2 * _H,), jnp.float32))


@jax.jit
def _gnn_sc(data_f, src0, dst0, src1, dst1, p0t, p1t, min0, max0, min1,
            max1):
    mesh = plsc.VectorSubcoreMesh(core_axis_name="c", subcore_axis_name="s",
                                  num_cores=_NC, num_subcores=_NS)
    f = pl.kernel(
        _sc_body,
        out_type=jax.ShapeDtypeStruct((_B * _N2 * _H,), jnp.float32),
        mesh=mesh,
        compiler_params=pltpu.CompilerParams(needs_layout_passes=False),
        scratch_types=[
            pltpu.VMEM((2048,), jnp.float32),
            pltpu.VMEM((_N2 * _L,), jnp.float32),
            pltpu.VMEM((_N2,), jnp.float32),
            pltpu.VMEM((_CHUNK,), jnp.int32),
            pltpu.VMEM((_CHUNK,), jnp.int32),
            pltpu.VMEM((_CHUNK,), jnp.int32),
            pltpu.VMEM((_CHUNK,), jnp.int32),
            pltpu.SemaphoreType.DMA,
            pltpu.SemaphoreType.DMA,
            pltpu.SemaphoreType.DMA,
            pltpu.SemaphoreType.DMA,
            pltpu.VMEM((128,), jnp.float32),
            pltpu.VMEM((128,), jnp.float32),
            pltpu.VMEM((_L * 17,), jnp.float32),
            pltpu.VMEM_SHARED(((_NS + 1) * 2048,), jnp.float32),
        ],
    )
    return f(data_f, src0, dst0, src1, dst1, p0t, p1t, min0, max0, min1,
             max1)


def _mm_body(h_ref, w_ref, b_ref, o_ref):
    o_ref[...] = lax.dot_general(
        h_ref[...], w_ref[...], (((1,), (1,)), ((), ())),
        preferred_element_type=jnp.float32) + b_ref[...]


@jax.jit
def _out_proj(h1, w_out, b_out2d):
    return pl.pallas_call(
        _mm_body,
        out_shape=jax.ShapeDtypeStruct((_B, _C), jnp.float32),
    )(h1, w_out, b_out2d)


def kernel(data, edge_index0, edge_index1, params0, params1,
           min0, max0, min1, max1, W_out, b_out):
    src0 = edge_index0[0].astype(jnp.int32)
    dst0 = edge_index0[1].astype(jnp.int32)
    src1 = edge_index1[0].astype(jnp.int32)
    dst1 = edge_index1[1].astype(jnp.int32)
    h1_flat = _gnn_sc(data.reshape(-1), src0, dst0, src1, dst1,
                      params0.T.reshape(-1), params1.T.reshape(-1),
                      min0, max0, min1, max1)
    h1 = h1_flat.reshape(_B, _N2 * _H)
    return _out_proj(h1, W_out, b_out.reshape(1, _C))
